# Initial kernel scaffold; baseline (speedup 1.0000x reference)
import functools

import jax
import jax.numpy as jnp
from jax import lax
from jax.experimental import pallas as pl
from jax.experimental.pallas import tpu as pltpu
from jax.experimental.pallas import tpu_sc as plsc

VOCAB = 100000
D = 64
T = 200
B = 4096
NC = 2
NS = 16
NW = NC * NS
ROWS_PER_W = B // NW
LANES = 16


def _emb_body(ids_hbm, tok_hbm, pos_hbm, out_hbm, idx_v, rows_v, pos_v, sem):
    wid = lax.axis_index("s") * NC + lax.axis_index("c")
    base_row = wid * ROWS_PER_W
    pltpu.sync_copy(pos_hbm, pos_v)

    def row_body(r, carry):
        row = base_row + r
        tok0 = row * T
        pltpu.sync_copy(ids_hbm.at[pl.ds(tok0, T)], idx_v)
        pltpu.async_copy(tok_hbm.at[idx_v], rows_v, sem).wait()

        def add_t(t, c):
            for j in range(D // LANES):
                sl = pl.ds(j * LANES, LANES)
                plsc.addupdate(rows_v.at[t, sl], pos_v[t, sl])
            return c

        lax.fori_loop(0, T, add_t, 0)
        pltpu.sync_copy(rows_v, out_hbm.at[pl.ds(tok0, T)])
        return carry

    lax.fori_loop(0, ROWS_PER_W, row_body, 0)


_emb_kernel = functools.partial(
    pl.kernel,
    out_type=jax.ShapeDtypeStruct((B * T, D), jnp.float32),
    mesh=plsc.VectorSubcoreMesh(core_axis_name="c", subcore_axis_name="s"),
    scratch_types=[
        pltpu.VMEM((T,), jnp.int32),
        pltpu.VMEM((T, D), jnp.float32),
        pltpu.VMEM((T, D), jnp.float32),
        pltpu.SemaphoreType.DMA,
    ],
)(_emb_body)


def kernel(input_ids, tok_table, pos_table):
    batch, block = input_ids.shape
    ids_flat = input_ids.reshape(-1).astype(jnp.int32)
    out = _emb_kernel(ids_flat, tok_table, pos_table)
    return out.reshape(batch, block, D)


# speedup vs baseline: 3.1046x; 3.1046x over previous
import functools

import jax
import jax.numpy as jnp
from jax import lax
from jax.experimental import pallas as pl
from jax.experimental.pallas import tpu as pltpu
from jax.experimental.pallas import tpu_sc as plsc

VOCAB = 100000
D = 64
T = 200
B = 4096
NC = 2
NS = 16
NW = NC * NS
ROWS_PER_W = B // NW
LANES = 16


def _emb_body(ids_hbm, tok_hbm, pos_hbm, out_hbm, idx_v, rows_v, pos_v, sem):
    wid = lax.axis_index("s") * NC + lax.axis_index("c")
    base_row = wid * ROWS_PER_W
    pltpu.sync_copy(pos_hbm, pos_v)

    def row_body(r, carry):
        row = base_row + r
        tok0 = row * T
        pltpu.sync_copy(ids_hbm.at[pl.ds(tok0, T)], idx_v)
        pltpu.async_copy(tok_hbm.at[idx_v], rows_v, sem).wait()

        def add_t(t, c):
            for j in range(D // LANES):
                sl = pl.ds(j * LANES, LANES)
                plsc.addupdate(rows_v.at[t, sl], pos_v[t, sl])
            return c

        lax.fori_loop(0, T, add_t, 0)
        pltpu.sync_copy(rows_v, out_hbm.at[pl.ds(tok0, T)])
        return carry

    lax.fori_loop(0, ROWS_PER_W, row_body, 0)


_emb_kernel = functools.partial(
    pl.kernel,
    out_type=jax.ShapeDtypeStruct((B * T, D), jnp.float32),
    mesh=plsc.VectorSubcoreMesh(core_axis_name="c", subcore_axis_name="s"),
    scratch_types=[
        pltpu.VMEM((T,), jnp.int32),
        pltpu.VMEM((T, D), jnp.float32),
        pltpu.VMEM((T, D), jnp.float32),
        pltpu.SemaphoreType.DMA,
    ],
    compiler_params=pltpu.CompilerParams(use_tc_tiling_on_sc=False),
)(_emb_body)


def kernel(input_ids, tok_table, pos_table):
    batch, block = input_ids.shape
    ids_flat = input_ids.reshape(-1).astype(jnp.int32)
    out = _emb_kernel(ids_flat, tok_table, pos_table)
    return out.reshape(batch, block, D)


# speedup vs baseline: 4.2263x; 1.3613x over previous
import functools

import jax
import jax.numpy as jnp
from jax import lax
from jax.experimental import pallas as pl
from jax.experimental.pallas import tpu as pltpu
from jax.experimental.pallas import tpu_sc as plsc

VOCAB = 100000
D = 64
T = 200
B = 4096
NC = 2
NS = 16
NW = NC * NS
ROWS_PER_W = B // NW
LANES = 16
C = 4
CT = C * T
NCHUNK = ROWS_PER_W // C
NSTEP = NCHUNK // 2


def _emb_body(ids_hbm, tok_hbm, pos_hbm, out_hbm,
              idx0, idx1, rows0, rows1, pos_v,
              gsem0, gsem1, wsem0, wsem1):
    wid = lax.axis_index("s") * NC + lax.axis_index("c")
    tok_base = wid * ROWS_PER_W * T
    pltpu.sync_copy(pos_hbm, pos_v)

    idx = (idx0, idx1)
    rows = (rows0, rows1)
    gsem = (gsem0, gsem1)
    wsem = (wsem0, wsem1)

    def add_pos(rbuf):
        def add_t(t, c):
            for j in range(D // LANES):
                sl = pl.ds(j * LANES, LANES)
                pv = pos_v[t, sl]
                for cc in range(C):
                    plsc.addupdate(rbuf.at[cc * T + t, sl], pv)
            return c
        lax.fori_loop(0, T, add_t, 0)

    pltpu.sync_copy(ids_hbm.at[pl.ds(tok_base, CT)], idx0)
    pltpu.async_copy(tok_hbm.at[idx0], rows0, gsem0)

    def step_body(s, carry):
        for b in range(2):
            g = 2 * s + b
            nb = 1 - b
            tok0 = tok_base + g * CT
            if b == 0:
                pltpu.sync_copy(ids_hbm.at[pl.ds(tok0 + CT, CT)], idx[nb])

                @pl.when(s > 0)
                def _wait_prev_write():
                    pltpu.make_async_copy(
                        rows[nb], out_hbm.at[pl.ds(tok0 - CT, CT)], wsem[nb]
                    ).wait()

                pltpu.async_copy(tok_hbm.at[idx[nb]], rows[nb], gsem[nb])
            else:
                @pl.when(s < NSTEP - 1)
                def _stage_next():
                    pltpu.sync_copy(ids_hbm.at[pl.ds(tok0 + CT, CT)], idx[nb])
                    pltpu.make_async_copy(
                        rows[nb], out_hbm.at[pl.ds(tok0 - CT, CT)], wsem[nb]
                    ).wait()
                    pltpu.async_copy(tok_hbm.at[idx[nb]], rows[nb], gsem[nb])

            pltpu.make_async_copy(tok_hbm.at[idx[b]], rows[b], gsem[b]).wait()
            add_pos(rows[b])
            pltpu.async_copy(rows[b], out_hbm.at[pl.ds(tok0, CT)], wsem[b])
        return carry

    lax.fori_loop(0, NSTEP, step_body, 0)

    pltpu.make_async_copy(
        rows0, out_hbm.at[pl.ds(tok_base + (NCHUNK - 2) * CT, CT)], wsem0
    ).wait()
    pltpu.make_async_copy(
        rows1, out_hbm.at[pl.ds(tok_base + (NCHUNK - 1) * CT, CT)], wsem1
    ).wait()


_emb_kernel = functools.partial(
    pl.kernel,
    out_type=jax.ShapeDtypeStruct((B * T, D), jnp.float32),
    mesh=plsc.VectorSubcoreMesh(core_axis_name="c", subcore_axis_name="s"),
    scratch_types=[
        pltpu.VMEM((CT,), jnp.int32),
        pltpu.VMEM((CT,), jnp.int32),
        pltpu.VMEM((CT, D), jnp.float32),
        pltpu.VMEM((CT, D), jnp.float32),
        pltpu.VMEM((T, D), jnp.float32),
        pltpu.SemaphoreType.DMA,
        pltpu.SemaphoreType.DMA,
        pltpu.SemaphoreType.DMA,
        pltpu.SemaphoreType.DMA,
    ],
    compiler_params=pltpu.CompilerParams(use_tc_tiling_on_sc=False),
)(_emb_body)


def kernel(input_ids, tok_table, pos_table):
    batch, block = input_ids.shape
    ids_flat = input_ids.reshape(-1).astype(jnp.int32)
    out = _emb_kernel(ids_flat, tok_table, pos_table)
    return out.reshape(batch, block, D)
